# Initial kernel scaffold; baseline (speedup 1.0000x reference)
import jax
import jax.numpy as jnp
from jax.experimental import pallas as pl


def kernel(x, W1, b1, W2, b2, ln_g, ln_b, Wg):
    raise NotImplementedError("write your pallas kernel here")


# speedup vs baseline: 3.3134x; 3.3134x over previous
import functools

import jax
import jax.numpy as jnp
from jax import lax
from jax.experimental import pallas as pl
from jax.experimental.pallas import tpu as pltpu
from jax.experimental.pallas import tpu_sc as plsc

_E = 8
_K = 2
_D = 1024
_H = 2048
_HB = 1024
_TB = 512
_RB = 256
_LN_EPS = 1e-6
_LANES = 128


def _gate_kernel(x_ref, wg_ref, idx_ref, w_ref, rank_ref, meta_ref, carry_ref):
    i = pl.program_id(0)

    @pl.when(i == 0)
    def _():
        carry_ref[...] = jnp.zeros_like(carry_ref)

    x = x_ref[...]
    scores = jnp.dot(x, wg_ref[...], preferred_element_type=jnp.float32)
    lanes = lax.broadcasted_iota(jnp.int32, scores.shape, 1)
    neg = jnp.float32(-1e30)
    s = jnp.where(lanes < _E, scores, neg)
    m1 = jnp.max(s, axis=1, keepdims=True)
    a1 = jnp.argmax(s, axis=1).astype(jnp.int32)
    s2 = jnp.where(lanes == a1[:, None], neg, s)
    m2 = jnp.max(s2, axis=1, keepdims=True)
    a2 = jnp.argmax(s2, axis=1).astype(jnp.int32)
    e2 = jnp.exp(m2 - m1)
    w1 = e2 / (1.0 + e2)
    w0 = 1.0 - w1

    oh = jnp.logical_or(lanes == a1[:, None], lanes == a2[:, None])
    oh = oh.astype(jnp.float32)
    row = lax.broadcasted_iota(jnp.int32, (_TB, _TB), 0)
    col = lax.broadcasted_iota(jnp.int32, (_TB, _TB), 1)
    tri = (col < row).astype(jnp.float32)
    cum = lax.dot_general(tri, oh, (((1,), (0,)), ((), ())),
                          preferred_element_type=jnp.float32)
    carry = carry_ref[0:1, :]
    rank_mat = cum + carry
    total = carry + jnp.sum(oh, axis=0, keepdims=True)
    carry_ref[0:1, :] = total

    r1 = jnp.sum(jnp.where(lanes == a1[:, None], rank_mat, 0.0), axis=1)
    r2 = jnp.sum(jnp.where(lanes == a2[:, None], rank_mat, 0.0), axis=1)
    idx_ref[...] = jnp.stack([a1, a2], axis=1)
    rank_ref[...] = jnp.stack([r1.astype(jnp.int32), r2.astype(jnp.int32)],
                              axis=1)
    w_ref[...] = jnp.concatenate([w0, w1], axis=1)

    pc = jnp.floor((total + (_RB - 1)) * (1.0 / _RB)) * _RB
    r128 = lax.broadcasted_iota(jnp.int32, (_LANES, _LANES), 0)
    c128 = lax.broadcasted_iota(jnp.int32, (_LANES, _LANES), 1)
    up = (r128 < c128).astype(jnp.float32)
    seg = lax.dot_general(pc, up, (((1,), (0,)), ((), ())),
                          preferred_element_type=jnp.float32)
    incl = seg + pc
    lane_r = lax.broadcasted_iota(jnp.int32, (1, _LANES), 1)
    bv = lane_r.astype(jnp.float32) * _RB
    acc = jnp.zeros((1, _LANES), jnp.float32)
    for e in range(_E):
        th = jnp.sum(jnp.where(lane_r == e, incl, 0.0), keepdims=True)
        acc = acc + (bv >= th).astype(jnp.float32)
    be_row = jnp.minimum(acc, float(_E - 1))
    meta = jnp.concatenate(
        [total, seg, incl, be_row,
         jnp.zeros((4, _LANES), jnp.float32)], axis=0)
    meta_ref[...] = meta.astype(jnp.int32)


def _dest_kernel(idx_ref, rank_ref, meta_ref, dest_ref):
    seg_row = meta_ref[1:2, :].astype(jnp.float32)
    lane_r = lax.broadcasted_iota(jnp.int32, (1, _LANES), 1)
    idxv = idx_ref[...]
    acc = jnp.zeros(idxv.shape, jnp.int32)
    for e in range(_E):
        th = jnp.sum(jnp.where(lane_r == e, seg_row, 0.0),
                     keepdims=True).astype(jnp.int32)
        acc = acc + jnp.where(idxv == e, th, 0)
    dest_ref[...] = acc + rank_ref[...]


def _ffn_kernel(be_ref, xs_ref, ws_ref, w1_ref, b1_ref, w2_ref, b2_ref,
                g_ref, lb_ref, ys_ref, acc_ref):
    del be_ref
    j = pl.program_id(1)
    nh = pl.num_programs(1)
    x = xs_ref[...]
    h = jnp.dot(x, w1_ref[0], preferred_element_type=jnp.float32) + b1_ref[0]
    h = 0.5 * h * (1.0 + lax.erf(h * 0.7071067811865476))
    part = jnp.dot(h, w2_ref[0], preferred_element_type=jnp.float32)

    @pl.when(j == 0)
    def _():
        acc_ref[...] = part + b2_ref[0]

    @pl.when(j != 0)
    def _():
        acc_ref[...] = acc_ref[...] + part

    @pl.when(j == nh - 1)
    def _():
        res = x + acc_ref[...]
        mu = jnp.mean(res, axis=1, keepdims=True)
        d = res - mu
        var = jnp.mean(d * d, axis=1, keepdims=True)
        norm = d / jnp.sqrt(var + _LN_EPS)
        y = norm * g_ref[0] + lb_ref[0]
        ys_ref[...] = y * ws_ref[0]


def _dispatch_body(tmask, apw, ch,
                   destp_hbm, wp_hbm, xf_hbm,
                   xs_hbm, ws_hbm,
                   dest_v, wv_v, tok_v, rows_v, sem):
    cid = lax.axis_index("c")
    sid = lax.axis_index("s")
    wid = sid * 2 + cid
    lane = lax.broadcasted_iota(jnp.int32, (16,), 0)

    for c in range(apw // ch):
        abase = wid * apw + c * ch
        pltpu.sync_copy(destp_hbm.at[pl.ds(abase, ch)], dest_v)
        pltpu.sync_copy(wp_hbm.at[pl.ds(abase, ch)], wv_v)
        for j in range(ch // 16):
            tok_v[pl.ds(j * 16, 16)] = (lane + (abase + j * 16)) & tmask
        pltpu.async_copy(xf_hbm.at[tok_v], rows_v, sem).wait()
        pltpu.async_copy(rows_v, xs_hbm.at[dest_v], sem).wait()
        pltpu.async_copy(wv_v, ws_hbm.at[dest_v], sem).wait()


def _combine_body(t, tpw, cht, ys_hbm, destp_hbm, y_hbm,
                  d0_v, d1_v, buf0_v, buf1_v, sem):
    cid = lax.axis_index("c")
    sid = lax.axis_index("s")
    wid = sid * 2 + cid

    for c in range(tpw // cht):
        tbase = wid * tpw + c * cht
        pltpu.sync_copy(destp_hbm.at[pl.ds(tbase, cht)], d0_v)
        pltpu.sync_copy(destp_hbm.at[pl.ds(t + tbase, cht)], d1_v)
        pltpu.async_copy(ys_hbm.at[d0_v], buf0_v, sem).wait()
        pltpu.async_copy(ys_hbm.at[d1_v], buf1_v, sem).wait()

        def add_row(r, carry):
            for dd in range(_D // 16):
                sl = pl.ds(dd * 16, 16)
                buf0_v[r, sl] = buf0_v[r, sl] + buf1_v[r, sl]
            return carry

        lax.fori_loop(0, cht, add_row, 0)
        pltpu.sync_copy(buf0_v, y_hbm.at[pl.ds(tbase, cht)])


def kernel(x, W1, b1, W2, b2, ln_g, ln_b, Wg):
    orig_shape = x.shape
    xf = x.reshape(-1, _D)
    T = xf.shape[0]
    A = T * _K
    nblk = A // _RB + _E
    nbe_pad = ((nblk + 15) // 16) * 16
    t2p = nblk * _RB

    wg_p = jnp.pad(Wg, ((0, 0), (0, _LANES - _E)))

    idx, w, rank, meta = pl.pallas_call(
        _gate_kernel,
        grid=(T // _TB,),
        in_specs=[
            pl.BlockSpec((_TB, _D), lambda i: (i, 0)),
            pl.BlockSpec((_D, _LANES), lambda i: (0, 0)),
        ],
        out_specs=[
            pl.BlockSpec((_TB, _K), lambda i: (i, 0)),
            pl.BlockSpec((_TB, _K), lambda i: (i, 0)),
            pl.BlockSpec((_TB, _K), lambda i: (i, 0)),
            pl.BlockSpec((8, _LANES), lambda i: (0, 0)),
        ],
        out_shape=[
            jax.ShapeDtypeStruct((T, _K), jnp.int32),
            jax.ShapeDtypeStruct((T, _K), jnp.float32),
            jax.ShapeDtypeStruct((T, _K), jnp.int32),
            jax.ShapeDtypeStruct((8, _LANES), jnp.int32),
        ],
        scratch_shapes=[pltpu.VMEM((8, _LANES), jnp.float32)],
        compiler_params=pltpu.CompilerParams(
            dimension_semantics=("arbitrary",)),
    )(xf, wg_p)

    dest = pl.pallas_call(
        _dest_kernel,
        grid=(1,),
        in_specs=[
            pl.BlockSpec((T, _K), lambda i: (0, 0)),
            pl.BlockSpec((T, _K), lambda i: (0, 0)),
            pl.BlockSpec((8, _LANES), lambda i: (0, 0)),
        ],
        out_specs=pl.BlockSpec((T, _K), lambda i: (0, 0)),
        out_shape=jax.ShapeDtypeStruct((T, _K), jnp.int32),
    )(idx, rank, meta)

    dest_p = dest.T.reshape(-1)
    w_p = w.T.reshape(-1)
    be = meta[3, :nbe_pad]

    nw = 32
    apw = A // nw
    ch = 64
    mesh = plsc.VectorSubcoreMesh(core_axis_name="c", subcore_axis_name="s")
    dispatch = functools.partial(
        pl.kernel,
        out_type=(
            jax.ShapeDtypeStruct((t2p, _D), jnp.float32),
            jax.ShapeDtypeStruct((t2p,), jnp.float32),
        ),
        mesh=mesh,
        scratch_types=[
            pltpu.VMEM((ch,), jnp.int32),
            pltpu.VMEM((ch,), jnp.float32),
            pltpu.VMEM((ch,), jnp.int32),
            pltpu.VMEM((ch, _D), jnp.float32),
            pltpu.SemaphoreType.DMA,
        ],
        compiler_params=pltpu.CompilerParams(needs_layout_passes=False),
    )(functools.partial(_dispatch_body, T - 1, apw, ch))
    xs, ws = dispatch(dest_p, w_p, xf)

    ws3 = ws.reshape(nblk, _RB, 1)
    b1r = b1.reshape(_E, 1, _H)
    b2r = b2.reshape(_E, 1, _D)
    ln_gr = ln_g.reshape(_E, 1, _D)
    ln_br = ln_b.reshape(_E, 1, _D)

    nh = _H // _HB
    ys = pl.pallas_call(
        _ffn_kernel,
        grid_spec=pltpu.PrefetchScalarGridSpec(
            num_scalar_prefetch=1,
            grid=(nblk, nh),
            in_specs=[
                pl.BlockSpec((_RB, _D), lambda i, j, be: (i, 0)),
                pl.BlockSpec((1, _RB, 1), lambda i, j, be: (i, 0, 0)),
                pl.BlockSpec((1, _D, _HB), lambda i, j, be: (be[i], 0, j)),
                pl.BlockSpec((1, 1, _HB), lambda i, j, be: (be[i], 0, j)),
                pl.BlockSpec((1, _HB, _D), lambda i, j, be: (be[i], j, 0)),
                pl.BlockSpec((1, 1, _D), lambda i, j, be: (be[i], 0, 0)),
                pl.BlockSpec((1, 1, _D), lambda i, j, be: (be[i], 0, 0)),
                pl.BlockSpec((1, 1, _D), lambda i, j, be: (be[i], 0, 0)),
            ],
            out_specs=pl.BlockSpec((_RB, _D), lambda i, j, be: (i, 0)),
            scratch_shapes=[pltpu.VMEM((_RB, _D), jnp.float32)],
        ),
        out_shape=jax.ShapeDtypeStruct((t2p, _D), jnp.float32),
        compiler_params=pltpu.CompilerParams(
            dimension_semantics=("arbitrary", "arbitrary")),
    )(be, xs, ws3, W1, b1r, W2, b2r, ln_gr, ln_br)

    tpw = T // nw
    cht = 32
    combine = functools.partial(
        pl.kernel,
        out_type=jax.ShapeDtypeStruct((T, _D), jnp.float32),
        mesh=mesh,
        scratch_types=[
            pltpu.VMEM((cht,), jnp.int32),
            pltpu.VMEM((cht,), jnp.int32),
            pltpu.VMEM((cht, _D), jnp.float32),
            pltpu.VMEM((cht, _D), jnp.float32),
            pltpu.SemaphoreType.DMA,
        ],
        compiler_params=pltpu.CompilerParams(needs_layout_passes=False),
    )(functools.partial(_combine_body, T, tpw, cht))
    y = combine(ys, dest_p)

    return y.reshape(orig_shape)


# speedup vs baseline: 3.7981x; 1.1463x over previous
import functools

import jax
import jax.numpy as jnp
from jax import lax
from jax.experimental import pallas as pl
from jax.experimental.pallas import tpu as pltpu
from jax.experimental.pallas import tpu_sc as plsc

_E = 8
_K = 2
_D = 1024
_H = 2048
_HB = 1024
_TB = 512
_RB = 256
_LN_EPS = 1e-6
_LANES = 128


def _gate_kernel(x_ref, wg_ref, idx_ref, w_ref, rank_ref, meta_ref, carry_ref):
    i = pl.program_id(0)

    @pl.when(i == 0)
    def _():
        carry_ref[...] = jnp.zeros_like(carry_ref)

    x = x_ref[...]
    scores = jnp.dot(x, wg_ref[...], preferred_element_type=jnp.float32)
    lanes = lax.broadcasted_iota(jnp.int32, scores.shape, 1)
    neg = jnp.float32(-1e30)
    s = jnp.where(lanes < _E, scores, neg)
    m1 = jnp.max(s, axis=1, keepdims=True)
    a1 = jnp.argmax(s, axis=1).astype(jnp.int32)
    s2 = jnp.where(lanes == a1[:, None], neg, s)
    m2 = jnp.max(s2, axis=1, keepdims=True)
    a2 = jnp.argmax(s2, axis=1).astype(jnp.int32)
    e2 = jnp.exp(m2 - m1)
    w1 = e2 / (1.0 + e2)
    w0 = 1.0 - w1

    oh = jnp.logical_or(lanes == a1[:, None], lanes == a2[:, None])
    oh = oh.astype(jnp.float32)
    row = lax.broadcasted_iota(jnp.int32, (_TB, _TB), 0)
    col = lax.broadcasted_iota(jnp.int32, (_TB, _TB), 1)
    tri = (col < row).astype(jnp.float32)
    cum = lax.dot_general(tri, oh, (((1,), (0,)), ((), ())),
                          preferred_element_type=jnp.float32)
    carry = carry_ref[0:1, :]
    rank_mat = cum + carry
    total = carry + jnp.sum(oh, axis=0, keepdims=True)
    carry_ref[0:1, :] = total

    r1 = jnp.sum(jnp.where(lanes == a1[:, None], rank_mat, 0.0), axis=1)
    r2 = jnp.sum(jnp.where(lanes == a2[:, None], rank_mat, 0.0), axis=1)
    idx_ref[...] = jnp.stack([a1, a2], axis=1)
    rank_ref[...] = jnp.stack([r1.astype(jnp.int32), r2.astype(jnp.int32)],
                              axis=1)
    w_ref[...] = jnp.concatenate([w0, w1], axis=1)

    pc = jnp.floor((total + (_RB - 1)) * (1.0 / _RB)) * _RB
    r128 = lax.broadcasted_iota(jnp.int32, (_LANES, _LANES), 0)
    c128 = lax.broadcasted_iota(jnp.int32, (_LANES, _LANES), 1)
    up = (r128 < c128).astype(jnp.float32)
    seg = lax.dot_general(pc, up, (((1,), (0,)), ((), ())),
                          preferred_element_type=jnp.float32)
    incl = seg + pc
    lane_r = lax.broadcasted_iota(jnp.int32, (1, _LANES), 1)
    bv = lane_r.astype(jnp.float32) * _RB
    acc = jnp.zeros((1, _LANES), jnp.float32)
    for e in range(_E):
        th = jnp.sum(jnp.where(lane_r == e, incl, 0.0), keepdims=True)
        acc = acc + (bv >= th).astype(jnp.float32)
    be_row = jnp.minimum(acc, float(_E - 1))
    meta = jnp.concatenate(
        [total, seg, incl, be_row,
         jnp.zeros((4, _LANES), jnp.float32)], axis=0)
    meta_ref[...] = meta.astype(jnp.int32)


def _dest_kernel(idx_ref, rank_ref, meta_ref, dest_ref):
    seg_row = meta_ref[1:2, :].astype(jnp.float32)
    lane_r = lax.broadcasted_iota(jnp.int32, (1, _LANES), 1)
    idxv = idx_ref[...]
    acc = jnp.zeros(idxv.shape, jnp.int32)
    for e in range(_E):
        th = jnp.sum(jnp.where(lane_r == e, seg_row, 0.0),
                     keepdims=True).astype(jnp.int32)
        acc = acc + jnp.where(idxv == e, th, 0)
    dest_ref[...] = acc + rank_ref[...]


def _ffn_kernel(be_ref, xs_ref, ws_ref, w1_ref, b1_ref, w2_ref, b2_ref,
                g_ref, lb_ref, ys_ref):
    del be_ref
    x = xs_ref[...]
    h = jnp.dot(x.astype(jnp.bfloat16), w1_ref[0],
                preferred_element_type=jnp.float32) + b1_ref[0]
    h = 0.5 * h * (1.0 + lax.erf(h * 0.7071067811865476))
    out = jnp.dot(h.astype(jnp.bfloat16), w2_ref[0],
                  preferred_element_type=jnp.float32) + b2_ref[0]
    res = x + out
    mu = jnp.mean(res, axis=1, keepdims=True)
    d = res - mu
    var = jnp.mean(d * d, axis=1, keepdims=True)
    norm = d / jnp.sqrt(var + _LN_EPS)
    y = norm * g_ref[0] + lb_ref[0]
    ys_ref[...] = y * ws_ref[0]


def _dispatch_body(tmask, apw, ch,
                   destp_hbm, wp_hbm, xf_hbm,
                   xs_hbm, ws_hbm,
                   dest_v, wv_v, tok_v, rows_v, sem):
    cid = lax.axis_index("c")
    sid = lax.axis_index("s")
    wid = sid * 2 + cid
    lane = lax.broadcasted_iota(jnp.int32, (16,), 0)

    for c in range(apw // ch):
        abase = wid * apw + c * ch
        pltpu.sync_copy(destp_hbm.at[pl.ds(abase, ch)], dest_v)
        pltpu.sync_copy(wp_hbm.at[pl.ds(abase, ch)], wv_v)
        for j in range(ch // 16):
            tok_v[pl.ds(j * 16, 16)] = (lane + (abase + j * 16)) & tmask
        pltpu.async_copy(xf_hbm.at[tok_v], rows_v, sem).wait()
        pltpu.async_copy(rows_v, xs_hbm.at[dest_v], sem).wait()
        pltpu.async_copy(wv_v, ws_hbm.at[dest_v], sem).wait()


def _combine_body(t, tpw, cht, ys_hbm, destp_hbm, y_hbm,
                  d0_v, d1_v, buf0_v, buf1_v, sem):
    cid = lax.axis_index("c")
    sid = lax.axis_index("s")
    wid = sid * 2 + cid

    for c in range(tpw // cht):
        tbase = wid * tpw + c * cht
        pltpu.sync_copy(destp_hbm.at[pl.ds(tbase, cht)], d0_v)
        pltpu.sync_copy(destp_hbm.at[pl.ds(t + tbase, cht)], d1_v)
        pltpu.async_copy(ys_hbm.at[d0_v], buf0_v, sem).wait()
        pltpu.async_copy(ys_hbm.at[d1_v], buf1_v, sem).wait()

        def add_row(r, carry):
            for dd in range(_D // 16):
                sl = pl.ds(dd * 16, 16)
                buf0_v[r, sl] = buf0_v[r, sl] + buf1_v[r, sl]
            return carry

        lax.fori_loop(0, cht, add_row, 0)
        pltpu.sync_copy(buf0_v, y_hbm.at[pl.ds(tbase, cht)])


def kernel(x, W1, b1, W2, b2, ln_g, ln_b, Wg):
    orig_shape = x.shape
    xf = x.reshape(-1, _D)
    T = xf.shape[0]
    A = T * _K
    nblk = A // _RB + _E
    nbe_pad = ((nblk + 15) // 16) * 16
    t2p = nblk * _RB

    wg_p = jnp.pad(Wg, ((0, 0), (0, _LANES - _E)))

    idx, w, rank, meta = pl.pallas_call(
        _gate_kernel,
        grid=(T // _TB,),
        in_specs=[
            pl.BlockSpec((_TB, _D), lambda i: (i, 0)),
            pl.BlockSpec((_D, _LANES), lambda i: (0, 0)),
        ],
        out_specs=[
            pl.BlockSpec((_TB, _K), lambda i: (i, 0)),
            pl.BlockSpec((_TB, _K), lambda i: (i, 0)),
            pl.BlockSpec((_TB, _K), lambda i: (i, 0)),
            pl.BlockSpec((8, _LANES), lambda i: (0, 0)),
        ],
        out_shape=[
            jax.ShapeDtypeStruct((T, _K), jnp.int32),
            jax.ShapeDtypeStruct((T, _K), jnp.float32),
            jax.ShapeDtypeStruct((T, _K), jnp.int32),
            jax.ShapeDtypeStruct((8, _LANES), jnp.int32),
        ],
        scratch_shapes=[pltpu.VMEM((8, _LANES), jnp.float32)],
        compiler_params=pltpu.CompilerParams(
            dimension_semantics=("arbitrary",)),
    )(xf, wg_p)

    dest = pl.pallas_call(
        _dest_kernel,
        grid=(1,),
        in_specs=[
            pl.BlockSpec((T, _K), lambda i: (0, 0)),
            pl.BlockSpec((T, _K), lambda i: (0, 0)),
            pl.BlockSpec((8, _LANES), lambda i: (0, 0)),
        ],
        out_specs=pl.BlockSpec((T, _K), lambda i: (0, 0)),
        out_shape=jax.ShapeDtypeStruct((T, _K), jnp.int32),
    )(idx, rank, meta)

    dest_p = dest.T.reshape(-1)
    w_p = w.T.reshape(-1)
    be = meta[3, :nbe_pad]

    nw = 32
    apw = A // nw
    ch = 64
    mesh = plsc.VectorSubcoreMesh(core_axis_name="c", subcore_axis_name="s")
    dispatch = functools.partial(
        pl.kernel,
        out_type=(
            jax.ShapeDtypeStruct((t2p, _D), jnp.float32),
            jax.ShapeDtypeStruct((t2p,), jnp.float32),
        ),
        mesh=mesh,
        scratch_types=[
            pltpu.VMEM((ch,), jnp.int32),
            pltpu.VMEM((ch,), jnp.float32),
            pltpu.VMEM((ch,), jnp.int32),
            pltpu.VMEM((ch, _D), jnp.float32),
            pltpu.SemaphoreType.DMA,
        ],
        compiler_params=pltpu.CompilerParams(needs_layout_passes=False),
    )(functools.partial(_dispatch_body, T - 1, apw, ch))
    xs, ws = dispatch(dest_p, w_p, xf)

    ws3 = ws.reshape(nblk, _RB, 1)
    w1b = W1.astype(jnp.bfloat16)
    w2b = W2.astype(jnp.bfloat16)
    b1r = b1.reshape(_E, 1, _H)
    b2r = b2.reshape(_E, 1, _D)
    ln_gr = ln_g.reshape(_E, 1, _D)
    ln_br = ln_b.reshape(_E, 1, _D)

    ys = pl.pallas_call(
        _ffn_kernel,
        grid_spec=pltpu.PrefetchScalarGridSpec(
            num_scalar_prefetch=1,
            grid=(nblk,),
            in_specs=[
                pl.BlockSpec((_RB, _D), lambda i, be: (i, 0)),
                pl.BlockSpec((1, _RB, 1), lambda i, be: (i, 0, 0)),
                pl.BlockSpec((1, _D, _H), lambda i, be: (be[i], 0, 0)),
                pl.BlockSpec((1, 1, _H), lambda i, be: (be[i], 0, 0)),
                pl.BlockSpec((1, _H, _D), lambda i, be: (be[i], 0, 0)),
                pl.BlockSpec((1, 1, _D), lambda i, be: (be[i], 0, 0)),
                pl.BlockSpec((1, 1, _D), lambda i, be: (be[i], 0, 0)),
                pl.BlockSpec((1, 1, _D), lambda i, be: (be[i], 0, 0)),
            ],
            out_specs=pl.BlockSpec((_RB, _D), lambda i, be: (i, 0)),
        ),
        out_shape=jax.ShapeDtypeStruct((t2p, _D), jnp.float32),
        compiler_params=pltpu.CompilerParams(
            dimension_semantics=("arbitrary",)),
    )(be, xs, ws3, w1b, b1r, w2b, b2r, ln_gr, ln_br)

    tpw = T // nw
    cht = 32
    combine = functools.partial(
        pl.kernel,
        out_type=jax.ShapeDtypeStruct((T, _D), jnp.float32),
        mesh=mesh,
        scratch_types=[
            pltpu.VMEM((cht,), jnp.int32),
            pltpu.VMEM((cht,), jnp.int32),
            pltpu.VMEM((cht, _D), jnp.float32),
            pltpu.VMEM((cht, _D), jnp.float32),
            pltpu.SemaphoreType.DMA,
        ],
        compiler_params=pltpu.CompilerParams(needs_layout_passes=False),
    )(functools.partial(_combine_body, T, tpw, cht))
    y = combine(ys, dest_p)

    return y.reshape(orig_shape)


# speedup vs baseline: 3.8660x; 1.0179x over previous
import functools

import jax
import jax.numpy as jnp
from jax import lax
from jax.experimental import pallas as pl
from jax.experimental.pallas import tpu as pltpu
from jax.experimental.pallas import tpu_sc as plsc

_E = 8
_K = 2
_D = 1024
_H = 2048
_HB = 1024
_TB = 512
_RB = 256
_LN_EPS = 1e-6
_LANES = 128


def _gate_kernel(x_ref, wg_ref, idx_ref, w_ref, rank_ref, meta_ref, carry_ref):
    i = pl.program_id(0)

    @pl.when(i == 0)
    def _():
        carry_ref[...] = jnp.zeros_like(carry_ref)

    x = x_ref[...]
    scores = jnp.dot(x, wg_ref[...], preferred_element_type=jnp.float32)
    lanes = lax.broadcasted_iota(jnp.int32, scores.shape, 1)
    neg = jnp.float32(-1e30)
    s = jnp.where(lanes < _E, scores, neg)
    m1 = jnp.max(s, axis=1, keepdims=True)
    a1 = jnp.argmax(s, axis=1).astype(jnp.int32)
    s2 = jnp.where(lanes == a1[:, None], neg, s)
    m2 = jnp.max(s2, axis=1, keepdims=True)
    a2 = jnp.argmax(s2, axis=1).astype(jnp.int32)
    e2 = jnp.exp(m2 - m1)
    w1 = e2 / (1.0 + e2)
    w0 = 1.0 - w1

    oh = jnp.logical_or(lanes == a1[:, None], lanes == a2[:, None])
    oh = oh.astype(jnp.float32)
    row = lax.broadcasted_iota(jnp.int32, (_TB, _TB), 0)
    col = lax.broadcasted_iota(jnp.int32, (_TB, _TB), 1)
    tri = (col < row).astype(jnp.float32)
    cum = lax.dot_general(tri, oh, (((1,), (0,)), ((), ())),
                          preferred_element_type=jnp.float32)
    carry = carry_ref[0:1, :]
    rank_mat = cum + carry
    total = carry + jnp.sum(oh, axis=0, keepdims=True)
    carry_ref[0:1, :] = total

    r1 = jnp.sum(jnp.where(lanes == a1[:, None], rank_mat, 0.0), axis=1)
    r2 = jnp.sum(jnp.where(lanes == a2[:, None], rank_mat, 0.0), axis=1)
    idx_ref[...] = jnp.stack([a1, a2], axis=1)
    rank_ref[...] = jnp.stack([r1.astype(jnp.int32), r2.astype(jnp.int32)],
                              axis=1)
    w_ref[...] = jnp.concatenate([w0, w1], axis=1)

    pc = jnp.floor((total + (_RB - 1)) * (1.0 / _RB)) * _RB
    r128 = lax.broadcasted_iota(jnp.int32, (_LANES, _LANES), 0)
    c128 = lax.broadcasted_iota(jnp.int32, (_LANES, _LANES), 1)
    up = (r128 < c128).astype(jnp.float32)
    seg = lax.dot_general(pc, up, (((1,), (0,)), ((), ())),
                          preferred_element_type=jnp.float32)
    incl = seg + pc
    lane_r = lax.broadcasted_iota(jnp.int32, (1, _LANES), 1)
    bv = lane_r.astype(jnp.float32) * _RB
    acc = jnp.zeros((1, _LANES), jnp.float32)
    for e in range(_E):
        th = jnp.sum(jnp.where(lane_r == e, incl, 0.0), keepdims=True)
        acc = acc + (bv >= th).astype(jnp.float32)
    be_row = jnp.minimum(acc, float(_E - 1))
    meta = jnp.concatenate(
        [total, seg, incl, be_row,
         jnp.zeros((4, _LANES), jnp.float32)], axis=0)
    meta_ref[...] = meta.astype(jnp.int32)


def _dest_kernel(idx_ref, rank_ref, meta_ref, dest_ref):
    seg_row = meta_ref[1:2, :].astype(jnp.float32)
    lane_r = lax.broadcasted_iota(jnp.int32, (1, _LANES), 1)
    idxv = idx_ref[...]
    acc = jnp.zeros(idxv.shape, jnp.int32)
    for e in range(_E):
        th = jnp.sum(jnp.where(lane_r == e, seg_row, 0.0),
                     keepdims=True).astype(jnp.int32)
        acc = acc + jnp.where(idxv == e, th, 0)
    dest_ref[...] = acc + rank_ref[...]


def _ffn_kernel(be_ref, xs_ref, ws_ref, w1_ref, b1_ref, w2_ref, b2_ref,
                g_ref, lb_ref, ys_ref):
    del be_ref
    x = xs_ref[...]
    h = jnp.dot(x.astype(jnp.bfloat16), w1_ref[0],
                preferred_element_type=jnp.float32) + b1_ref[0]
    h = 0.5 * h * (1.0 + lax.erf(h * 0.7071067811865476))
    out = jnp.dot(h.astype(jnp.bfloat16), w2_ref[0],
                  preferred_element_type=jnp.float32) + b2_ref[0]
    res = x + out
    mu = jnp.mean(res, axis=1, keepdims=True)
    d = res - mu
    var = jnp.mean(d * d, axis=1, keepdims=True)
    norm = d / jnp.sqrt(var + _LN_EPS)
    y = norm * g_ref[0] + lb_ref[0]
    ys_ref[...] = y * ws_ref[0]


def _dispatch_body(tmask, apw, ch,
                   destp_hbm, wp_hbm, xf_hbm,
                   xs_hbm, ws_hbm,
                   dest_v0, dest_v1, wv_v0, wv_v1, rows_v0, rows_v1,
                   sem0, sem1):
    cid = lax.axis_index("c")
    sid = lax.axis_index("s")
    wid = sid * 2 + cid
    bufs = ((dest_v0, wv_v0, rows_v0, sem0), (dest_v1, wv_v1, rows_v1, sem1))
    pending = [None, None]

    for c in range(apw // ch):
        b = c & 1
        dest_v, wv_v, rows_v, sem = bufs[b]
        if pending[b] is not None:
            for d in pending[b]:
                d.wait()
        abase = wid * apw + c * ch
        tok0 = pl.multiple_of(abase & tmask, 8)
        pltpu.sync_copy(destp_hbm.at[pl.ds(abase, ch)], dest_v)
        pltpu.sync_copy(wp_hbm.at[pl.ds(abase, ch)], wv_v)
        pltpu.sync_copy(xf_hbm.at[pl.ds(tok0, ch)], rows_v)
        d1 = pltpu.async_copy(rows_v, xs_hbm.at[dest_v], sem)
        d2 = pltpu.async_copy(wv_v, ws_hbm.at[dest_v], sem)
        pending[b] = (d1, d2)
    for p in pending:
        if p is not None:
            for d in p:
                d.wait()


def _combine_body(t, tpw, cht, ys_hbm, destp_hbm, y_hbm,
                  d0_v0, d0_v1, d1_v0, d1_v1,
                  buf0_v0, buf0_v1, buf1_v0, buf1_v1,
                  gsem0, gsem1, ssem0, ssem1):
    cid = lax.axis_index("c")
    sid = lax.axis_index("s")
    wid = sid * 2 + cid
    bufs = ((d0_v0, d1_v0, buf0_v0, buf1_v0, gsem0, ssem0),
            (d0_v1, d1_v1, buf0_v1, buf1_v1, gsem1, ssem1))
    pending = [None, None]

    for c in range(tpw // cht):
        b = c & 1
        d0_v, d1_v, buf0, buf1, gsem, ssem = bufs[b]
        if pending[b] is not None:
            pending[b].wait()
        tbase = wid * tpw + c * cht
        pltpu.sync_copy(destp_hbm.at[pl.ds(tbase, cht)], d0_v)
        pltpu.sync_copy(destp_hbm.at[pl.ds(t + tbase, cht)], d1_v)
        g0 = pltpu.async_copy(ys_hbm.at[d0_v], buf0, gsem)
        g1 = pltpu.async_copy(ys_hbm.at[d1_v], buf1, gsem)
        g0.wait()
        g1.wait()

        def add_row(r, carry):
            for dd in range(_D // 16):
                sl = pl.ds(dd * 16, 16)
                buf0[r, sl] = buf0[r, sl] + buf1[r, sl]
            return carry

        lax.fori_loop(0, cht, add_row, 0)
        pending[b] = pltpu.async_copy(
            buf0, y_hbm.at[pl.ds(tbase, cht)], ssem)
    for p in pending:
        if p is not None:
            p.wait()


def kernel(x, W1, b1, W2, b2, ln_g, ln_b, Wg):
    orig_shape = x.shape
    xf = x.reshape(-1, _D)
    T = xf.shape[0]
    A = T * _K
    nblk = A // _RB + _E
    nbe_pad = ((nblk + 15) // 16) * 16
    t2p = nblk * _RB

    wg_p = jnp.pad(Wg, ((0, 0), (0, _LANES - _E)))

    idx, w, rank, meta = pl.pallas_call(
        _gate_kernel,
        grid=(T // _TB,),
        in_specs=[
            pl.BlockSpec((_TB, _D), lambda i: (i, 0)),
            pl.BlockSpec((_D, _LANES), lambda i: (0, 0)),
        ],
        out_specs=[
            pl.BlockSpec((_TB, _K), lambda i: (i, 0)),
            pl.BlockSpec((_TB, _K), lambda i: (i, 0)),
            pl.BlockSpec((_TB, _K), lambda i: (i, 0)),
            pl.BlockSpec((8, _LANES), lambda i: (0, 0)),
        ],
        out_shape=[
            jax.ShapeDtypeStruct((T, _K), jnp.int32),
            jax.ShapeDtypeStruct((T, _K), jnp.float32),
            jax.ShapeDtypeStruct((T, _K), jnp.int32),
            jax.ShapeDtypeStruct((8, _LANES), jnp.int32),
        ],
        scratch_shapes=[pltpu.VMEM((8, _LANES), jnp.float32)],
        compiler_params=pltpu.CompilerParams(
            dimension_semantics=("arbitrary",)),
    )(xf, wg_p)

    dest = pl.pallas_call(
        _dest_kernel,
        grid=(1,),
        in_specs=[
            pl.BlockSpec((T, _K), lambda i: (0, 0)),
            pl.BlockSpec((T, _K), lambda i: (0, 0)),
            pl.BlockSpec((8, _LANES), lambda i: (0, 0)),
        ],
        out_specs=pl.BlockSpec((T, _K), lambda i: (0, 0)),
        out_shape=jax.ShapeDtypeStruct((T, _K), jnp.int32),
    )(idx, rank, meta)

    dest_p = dest.T.reshape(-1)
    w_p = w.T.reshape(-1)
    be = meta[3, :nbe_pad]

    nw = 32
    apw = A // nw
    ch = 32
    mesh = plsc.VectorSubcoreMesh(core_axis_name="c", subcore_axis_name="s")
    dispatch = functools.partial(
        pl.kernel,
        out_type=(
            jax.ShapeDtypeStruct((t2p, _D), jnp.float32),
            jax.ShapeDtypeStruct((t2p,), jnp.float32),
        ),
        mesh=mesh,
        scratch_types=[
            pltpu.VMEM((ch,), jnp.int32),
            pltpu.VMEM((ch,), jnp.int32),
            pltpu.VMEM((ch,), jnp.float32),
            pltpu.VMEM((ch,), jnp.float32),
            pltpu.VMEM((ch, _D), jnp.float32),
            pltpu.VMEM((ch, _D), jnp.float32),
            pltpu.SemaphoreType.DMA,
            pltpu.SemaphoreType.DMA,
        ],
        compiler_params=pltpu.CompilerParams(needs_layout_passes=False),
    )(functools.partial(_dispatch_body, T - 1, apw, ch))
    xs, ws = dispatch(dest_p, w_p, xf)

    ws3 = ws.reshape(nblk, _RB, 1)
    w1b = W1.astype(jnp.bfloat16)
    w2b = W2.astype(jnp.bfloat16)
    b1r = b1.reshape(_E, 1, _H)
    b2r = b2.reshape(_E, 1, _D)
    ln_gr = ln_g.reshape(_E, 1, _D)
    ln_br = ln_b.reshape(_E, 1, _D)

    ys = pl.pallas_call(
        _ffn_kernel,
        grid_spec=pltpu.PrefetchScalarGridSpec(
            num_scalar_prefetch=1,
            grid=(nblk,),
            in_specs=[
                pl.BlockSpec((_RB, _D), lambda i, be: (i, 0)),
                pl.BlockSpec((1, _RB, 1), lambda i, be: (i, 0, 0)),
                pl.BlockSpec((1, _D, _H), lambda i, be: (be[i], 0, 0)),
                pl.BlockSpec((1, 1, _H), lambda i, be: (be[i], 0, 0)),
                pl.BlockSpec((1, _H, _D), lambda i, be: (be[i], 0, 0)),
                pl.BlockSpec((1, 1, _D), lambda i, be: (be[i], 0, 0)),
                pl.BlockSpec((1, 1, _D), lambda i, be: (be[i], 0, 0)),
                pl.BlockSpec((1, 1, _D), lambda i, be: (be[i], 0, 0)),
            ],
            out_specs=pl.BlockSpec((_RB, _D), lambda i, be: (i, 0)),
        ),
        out_shape=jax.ShapeDtypeStruct((t2p, _D), jnp.float32),
        compiler_params=pltpu.CompilerParams(
            dimension_semantics=("arbitrary",)),
    )(be, xs, ws3, w1b, b1r, w2b, b2r, ln_gr, ln_br)

    tpw = T // nw
    cht = 16
    combine = functools.partial(
        pl.kernel,
        out_type=jax.ShapeDtypeStruct((T, _D), jnp.float32),
        mesh=mesh,
        scratch_types=[
            pltpu.VMEM((cht,), jnp.int32),
            pltpu.VMEM((cht,), jnp.int32),
            pltpu.VMEM((cht,), jnp.int32),
            pltpu.VMEM((cht,), jnp.int32),
            pltpu.VMEM((cht, _D), jnp.float32),
            pltpu.VMEM((cht, _D), jnp.float32),
            pltpu.VMEM((cht, _D), jnp.float32),
            pltpu.VMEM((cht, _D), jnp.float32),
            pltpu.SemaphoreType.DMA,
            pltpu.SemaphoreType.DMA,
            pltpu.SemaphoreType.DMA,
            pltpu.SemaphoreType.DMA,
        ],
        compiler_params=pltpu.CompilerParams(needs_layout_passes=False),
    )(functools.partial(_combine_body, T, tpw, cht))
    y = combine(ys, dest_p)

    return y.reshape(orig_shape)


# speedup vs baseline: 4.0282x; 1.0420x over previous
import functools

import jax
import jax.numpy as jnp
from jax import lax
from jax.experimental import pallas as pl
from jax.experimental.pallas import tpu as pltpu
from jax.experimental.pallas import tpu_sc as plsc

_E = 8
_K = 2
_D = 1024
_H = 2048
_HB = 1024
_TB = 512
_RB = 256
_LN_EPS = 1e-6
_LANES = 128


def _gate_kernel(x_ref, wg_ref, w_ref, dest_ref, meta_ref,
                 carry_ref, idx_s, rank_s):
    i = pl.program_id(0)
    nsteps = pl.num_programs(0)

    @pl.when(i == 0)
    def _():
        carry_ref[...] = jnp.zeros_like(carry_ref)

    @pl.when(i < nsteps - 1)
    def _gate_step():
        x = x_ref[...]
        scores = jnp.dot(x, wg_ref[...], preferred_element_type=jnp.float32)
        lanes = lax.broadcasted_iota(jnp.int32, scores.shape, 1)
        neg = jnp.float32(-1e30)
        s = jnp.where(lanes < _E, scores, neg)
        m1 = jnp.max(s, axis=1, keepdims=True)
        a1 = jnp.argmax(s, axis=1).astype(jnp.int32)
        s2 = jnp.where(lanes == a1[:, None], neg, s)
        m2 = jnp.max(s2, axis=1, keepdims=True)
        a2 = jnp.argmax(s2, axis=1).astype(jnp.int32)
        e2 = jnp.exp(m2 - m1)
        w1 = e2 / (1.0 + e2)
        w0 = 1.0 - w1

        oh = jnp.logical_or(lanes == a1[:, None], lanes == a2[:, None])
        oh = oh.astype(jnp.float32)
        row = lax.broadcasted_iota(jnp.int32, (_TB, _TB), 0)
        col = lax.broadcasted_iota(jnp.int32, (_TB, _TB), 1)
        tri = (col < row).astype(jnp.float32)
        cum = lax.dot_general(tri, oh, (((1,), (0,)), ((), ())),
                              preferred_element_type=jnp.float32)
        carry = carry_ref[0:1, :]
        rank_mat = cum + carry
        carry_ref[0:1, :] = carry + jnp.sum(oh, axis=0, keepdims=True)

        r1 = jnp.sum(jnp.where(lanes == a1[:, None], rank_mat, 0.0), axis=1)
        r2 = jnp.sum(jnp.where(lanes == a2[:, None], rank_mat, 0.0), axis=1)
        base = pl.multiple_of(i * _TB, _TB)
        idx_s[pl.ds(base, _TB), :] = jnp.stack([a1, a2], axis=1)
        rank_s[pl.ds(base, _TB), :] = jnp.stack(
            [r1.astype(jnp.int32), r2.astype(jnp.int32)], axis=1)
        w_ref[...] = jnp.concatenate([w0, w1], axis=1)

    @pl.when(i == nsteps - 1)
    def _routing_step():
        total = carry_ref[0:1, :]
        pc = jnp.floor((total + (_RB - 1)) * (1.0 / _RB)) * _RB
        r128 = lax.broadcasted_iota(jnp.int32, (_LANES, _LANES), 0)
        c128 = lax.broadcasted_iota(jnp.int32, (_LANES, _LANES), 1)
        up = (r128 < c128).astype(jnp.float32)
        seg = lax.dot_general(pc, up, (((1,), (0,)), ((), ())),
                              preferred_element_type=jnp.float32)
        incl = seg + pc
        lane_r = lax.broadcasted_iota(jnp.int32, (1, _LANES), 1)
        bv = lane_r.astype(jnp.float32) * _RB
        acc = jnp.zeros((1, _LANES), jnp.float32)
        for e in range(_E):
            th = jnp.sum(jnp.where(lane_r == e, incl, 0.0), keepdims=True)
            acc = acc + (bv >= th).astype(jnp.float32)
        tot_pad = jnp.sum(jnp.where(lane_r == _E - 1, incl, 0.0),
                          keepdims=True)
        lastne = jnp.max(jnp.where(pc > 0.0, lane_r.astype(jnp.float32), 0.0),
                         keepdims=True)
        used = bv < tot_pad
        be_row = jnp.where(used, jnp.minimum(acc, float(_E - 1)),
                           lastne + float(_E))
        meta = jnp.concatenate(
            [total, seg, incl, be_row,
             jnp.zeros((4, _LANES), jnp.float32)], axis=0)
        meta_ref[...] = meta.astype(jnp.int32)

        idxv = idx_s[...]
        dacc = jnp.zeros(idxv.shape, jnp.int32)
        for e in range(_E):
            th = jnp.sum(jnp.where(lane_r == e, seg, 0.0),
                         keepdims=True).astype(jnp.int32)
            dacc = dacc + jnp.where(idxv == e, th, 0)
        dest_ref[...] = dacc + rank_s[...]


def _ffn_kernel(be_ref, xs_ref, ws_ref, w1_ref, b1_ref, w2_ref, b2_ref,
                g_ref, lb_ref, ys_ref):
    i = pl.program_id(0)

    @pl.when(be_ref[i] < _E)
    def _():
        x = xs_ref[...]
        h = jnp.dot(x.astype(jnp.bfloat16), w1_ref[0],
                    preferred_element_type=jnp.float32) + b1_ref[0]
        h = 0.5 * h * (1.0 + lax.erf(h * 0.7071067811865476))
        out = jnp.dot(h.astype(jnp.bfloat16), w2_ref[0],
                      preferred_element_type=jnp.float32) + b2_ref[0]
        res = x + out
        mu = jnp.mean(res, axis=1, keepdims=True)
        d = res - mu
        var = jnp.mean(d * d, axis=1, keepdims=True)
        norm = d / jnp.sqrt(var + _LN_EPS)
        y = norm * g_ref[0] + lb_ref[0]
        ys_ref[...] = y * ws_ref[0]


def _dispatch_body(tmask, apw, ch,
                   destp_hbm, wp_hbm, xf_hbm,
                   xs_hbm, ws_hbm,
                   dest_v0, dest_v1, wv_v0, wv_v1, rows_v0, rows_v1,
                   sem0, sem1):
    cid = lax.axis_index("c")
    sid = lax.axis_index("s")
    wid = sid * 2 + cid
    bufs = ((dest_v0, wv_v0, rows_v0, sem0), (dest_v1, wv_v1, rows_v1, sem1))
    pending = [None, None]

    for c in range(apw // ch):
        b = c & 1
        dest_v, wv_v, rows_v, sem = bufs[b]
        if pending[b] is not None:
            for d in pending[b]:
                d.wait()
        abase = wid * apw + c * ch
        tok0 = pl.multiple_of(abase & tmask, 8)
        pltpu.sync_copy(destp_hbm.at[pl.ds(abase, ch)], dest_v)
        pltpu.sync_copy(wp_hbm.at[pl.ds(abase, ch)], wv_v)
        pltpu.sync_copy(xf_hbm.at[pl.ds(tok0, ch)], rows_v)
        d1 = pltpu.async_copy(rows_v, xs_hbm.at[dest_v], sem)
        d2 = pltpu.async_copy(wv_v, ws_hbm.at[dest_v], sem)
        pending[b] = (d1, d2)
    for p in pending:
        if p is not None:
            for d in p:
                d.wait()


def _combine_body(t, tpw, cht, ys_hbm, destp_hbm, y_hbm,
                  d0_v0, d0_v1, d1_v0, d1_v1,
                  buf0_v0, buf0_v1, buf1_v0, buf1_v1,
                  gsem0, gsem1, ssem0, ssem1):
    cid = lax.axis_index("c")
    sid = lax.axis_index("s")
    wid = sid * 2 + cid
    bufs = ((d0_v0, d1_v0, buf0_v0, buf1_v0, gsem0, ssem0),
            (d0_v1, d1_v1, buf0_v1, buf1_v1, gsem1, ssem1))
    pending = [None, None]

    for c in range(tpw // cht):
        b = c & 1
        d0_v, d1_v, buf0, buf1, gsem, ssem = bufs[b]
        if pending[b] is not None:
            pending[b].wait()
        tbase = wid * tpw + c * cht
        pltpu.sync_copy(destp_hbm.at[pl.ds(tbase, cht)], d0_v)
        pltpu.sync_copy(destp_hbm.at[pl.ds(t + tbase, cht)], d1_v)
        g0 = pltpu.async_copy(ys_hbm.at[d0_v], buf0, gsem)
        g1 = pltpu.async_copy(ys_hbm.at[d1_v], buf1, gsem)
        g0.wait()
        g1.wait()

        def add_row(r, carry):
            for dd in range(_D // 16):
                sl = pl.ds(dd * 16, 16)
                buf0[r, sl] = buf0[r, sl] + buf1[r, sl]
            return carry

        lax.fori_loop(0, cht, add_row, 0)
        pending[b] = pltpu.async_copy(
            buf0, y_hbm.at[pl.ds(tbase, cht)], ssem)
    for p in pending:
        if p is not None:
            p.wait()


def kernel(x, W1, b1, W2, b2, ln_g, ln_b, Wg):
    orig_shape = x.shape
    xf = x.reshape(-1, _D)
    T = xf.shape[0]
    A = T * _K
    nblk = A // _RB + _E
    nbe_pad = ((nblk + 15) // 16) * 16
    t2p = nblk * _RB

    wg_p = jnp.pad(Wg, ((0, 0), (0, _LANES - _E)))

    ngate = T // _TB
    w, dest, meta = pl.pallas_call(
        _gate_kernel,
        grid=(ngate + 1,),
        in_specs=[
            pl.BlockSpec((_TB, _D), lambda i: (jnp.minimum(i, ngate - 1), 0)),
            pl.BlockSpec((_D, _LANES), lambda i: (0, 0)),
        ],
        out_specs=[
            pl.BlockSpec((_TB, _K), lambda i: (jnp.minimum(i, ngate - 1), 0)),
            pl.BlockSpec((T, _K), lambda i: (0, 0)),
            pl.BlockSpec((8, _LANES), lambda i: (0, 0)),
        ],
        out_shape=[
            jax.ShapeDtypeStruct((T, _K), jnp.float32),
            jax.ShapeDtypeStruct((T, _K), jnp.int32),
            jax.ShapeDtypeStruct((8, _LANES), jnp.int32),
        ],
        scratch_shapes=[
            pltpu.VMEM((8, _LANES), jnp.float32),
            pltpu.VMEM((T, _K), jnp.int32),
            pltpu.VMEM((T, _K), jnp.int32),
        ],
        compiler_params=pltpu.CompilerParams(
            dimension_semantics=("arbitrary",)),
    )(xf, wg_p)

    dest_p = dest.T.reshape(-1)
    w_p = w.T.reshape(-1)
    be = meta[3, :nbe_pad]

    nw = 32
    apw = A // nw
    ch = 32
    mesh = plsc.VectorSubcoreMesh(core_axis_name="c", subcore_axis_name="s")
    dispatch = functools.partial(
        pl.kernel,
        out_type=(
            jax.ShapeDtypeStruct((t2p, _D), jnp.float32),
            jax.ShapeDtypeStruct((t2p,), jnp.float32),
        ),
        mesh=mesh,
        scratch_types=[
            pltpu.VMEM((ch,), jnp.int32),
            pltpu.VMEM((ch,), jnp.int32),
            pltpu.VMEM((ch,), jnp.float32),
            pltpu.VMEM((ch,), jnp.float32),
            pltpu.VMEM((ch, _D), jnp.float32),
            pltpu.VMEM((ch, _D), jnp.float32),
            pltpu.SemaphoreType.DMA,
            pltpu.SemaphoreType.DMA,
        ],
        compiler_params=pltpu.CompilerParams(needs_layout_passes=False),
    )(functools.partial(_dispatch_body, T - 1, apw, ch))
    xs, ws = dispatch(dest_p, w_p, xf)

    ws3 = ws.reshape(nblk, _RB, 1)
    w1b = W1.astype(jnp.bfloat16)
    w2b = W2.astype(jnp.bfloat16)
    b1r = b1.reshape(_E, 1, _H)
    b2r = b2.reshape(_E, 1, _D)
    ln_gr = ln_g.reshape(_E, 1, _D)
    ln_br = ln_b.reshape(_E, 1, _D)

    ys = pl.pallas_call(
        _ffn_kernel,
        grid_spec=pltpu.PrefetchScalarGridSpec(
            num_scalar_prefetch=1,
            grid=(nblk,),
            in_specs=[
                pl.BlockSpec(
                    (_RB, _D),
                    lambda i, be: (jnp.where(be[i] < _E, i, 0), 0)),
                pl.BlockSpec(
                    (1, _RB, 1),
                    lambda i, be: (jnp.where(be[i] < _E, i, 0), 0, 0)),
                pl.BlockSpec((1, _D, _H), lambda i, be: (be[i] & 7, 0, 0)),
                pl.BlockSpec((1, 1, _H), lambda i, be: (be[i] & 7, 0, 0)),
                pl.BlockSpec((1, _H, _D), lambda i, be: (be[i] & 7, 0, 0)),
                pl.BlockSpec((1, 1, _D), lambda i, be: (be[i] & 7, 0, 0)),
                pl.BlockSpec((1, 1, _D), lambda i, be: (be[i] & 7, 0, 0)),
                pl.BlockSpec((1, 1, _D), lambda i, be: (be[i] & 7, 0, 0)),
            ],
            out_specs=pl.BlockSpec((_RB, _D), lambda i, be: (i, 0)),
        ),
        out_shape=jax.ShapeDtypeStruct((t2p, _D), jnp.float32),
        compiler_params=pltpu.CompilerParams(
            dimension_semantics=("arbitrary",)),
    )(be, xs, ws3, w1b, b1r, w2b, b2r, ln_gr, ln_br)

    tpw = T // nw
    cht = 16
    combine = functools.partial(
        pl.kernel,
        out_type=jax.ShapeDtypeStruct((T, _D), jnp.float32),
        mesh=mesh,
        scratch_types=[
            pltpu.VMEM((cht,), jnp.int32),
            pltpu.VMEM((cht,), jnp.int32),
            pltpu.VMEM((cht,), jnp.int32),
            pltpu.VMEM((cht,), jnp.int32),
            pltpu.VMEM((cht, _D), jnp.float32),
            pltpu.VMEM((cht, _D), jnp.float32),
            pltpu.VMEM((cht, _D), jnp.float32),
            pltpu.VMEM((cht, _D), jnp.float32),
            pltpu.SemaphoreType.DMA,
            pltpu.SemaphoreType.DMA,
            pltpu.SemaphoreType.DMA,
            pltpu.SemaphoreType.DMA,
        ],
        compiler_params=pltpu.CompilerParams(needs_layout_passes=False),
    )(functools.partial(_combine_body, T, tpw, cht))
    y = combine(ys, dest_p)

    return y.reshape(orig_shape)
